# baseline (speedup 1.0000x reference)
import functools

import jax
import jax.numpy as jnp
from jax import lax
from jax.experimental import pallas as pl
from jax.experimental.pallas import tpu as pltpu
from jax.experimental.pallas import tpu_sc as plsc


def _build_gather(batch, seq, d, seq_pad):
    info = plsc.get_sparse_core_info()
    nc, ns = info.num_cores, info.num_subcores
    nw = nc * ns
    assert batch % nw == 0
    per_w = batch // nw
    assert per_w % 2 == 0
    n_full = seq // 8
    n_rem = seq - 8 * n_full

    mesh = plsc.VectorSubcoreMesh(core_axis_name="c", subcore_axis_name="s")

    @functools.partial(
        pl.kernel,
        mesh=mesh,
        out_type=jax.ShapeDtypeStruct((batch, seq, d), jnp.float32),
        scratch_types=[
            pltpu.VMEM((per_w * seq_pad,), jnp.int32),
            pltpu.VMEM((seq_pad, d), jnp.float32),
            pltpu.VMEM((seq_pad, d), jnp.float32),
            pltpu.SemaphoreType.DMA,
            pltpu.SemaphoreType.DMA,
            pltpu.SemaphoreType.DMA,
            pltpu.SemaphoreType.DMA,
        ],
        compiler_params=pltpu.CompilerParams(use_tc_tiling_on_sc=True),
    )
    def gather_kernel(idx_hbm, table_hbm, out_hbm, idx_v, buf_a, buf_b,
                      gsem_a, gsem_b, osem_a, osem_b):
        wid = lax.axis_index("s") * nc + lax.axis_index("c")
        b0 = wid * per_w

        pltpu.sync_copy(idx_hbm.at[pl.ds(b0 * seq_pad, per_w * seq_pad)],
                        idx_v)

        bufs = ((buf_a, gsem_a, osem_a), (buf_b, gsem_b, osem_b))

        def gather_src(i):
            return table_hbm.at[idx_v.at[pl.ds(i * seq_pad, seq_pad)]]

        def start_gather(i, buf, gsem):
            pltpu.async_copy(gather_src(i), buf, gsem)

        def wait_gather(i, buf, gsem):
            pltpu.make_async_copy(gather_src(i), buf, gsem).wait()

        def start_writes(i, buf, osem):
            b = b0 + i
            for t in range(n_full):
                pltpu.async_copy(buf.at[pl.ds(8 * t, 8)],
                                 out_hbm.at[b, pl.ds(8 * t, 8)], osem)
            if n_rem:
                pltpu.async_copy(buf.at[pl.ds(8 * n_full, n_rem)],
                                 out_hbm.at[b, pl.ds(8 * n_full, n_rem)],
                                 osem)

        def wait_writes(buf, osem):
            for t in range(n_full):
                pltpu.make_async_copy(buf.at[pl.ds(0, 8)],
                                      out_hbm.at[b0, pl.ds(0, 8)],
                                      osem).wait()
            if n_rem:
                pltpu.make_async_copy(buf.at[pl.ds(0, n_rem)],
                                      out_hbm.at[b0, pl.ds(0, n_rem)],
                                      osem).wait()

        start_gather(0, buf_a, gsem_a)
        start_gather(1, buf_b, gsem_b)

        def body(p, carry):
            for b, (buf, gsem, osem) in enumerate(bufs):
                i = 2 * p + b
                wait_gather(i, buf, gsem)
                start_writes(i, buf, osem)
                wait_writes(buf, osem)
                start_gather(i + 2, buf, gsem)
            return carry

        lax.fori_loop(0, per_w // 2 - 1, body, 0)

        for b, (buf, gsem, osem) in enumerate(bufs):
            i = per_w - 2 + b
            wait_gather(i, buf, gsem)
            start_writes(i, buf, osem)
            wait_writes(buf, osem)

    return gather_kernel


def kernel(indices, bank_embedding_weight):
    b, s = indices.shape
    v, d = bank_embedding_weight.shape
    seq_pad = (s + 7) // 8 * 8
    idx_pad = jnp.pad(indices.astype(jnp.int32), ((0, 0), (0, seq_pad - s)))
    flat = idx_pad.reshape(b * seq_pad)
    return _build_gather(b, s, d, seq_pad)(flat, bank_embedding_weight)


# speedup vs baseline: 1.8923x; 1.8923x over previous
import functools

import jax
import jax.numpy as jnp
from jax import lax
from jax.experimental import pallas as pl
from jax.experimental.pallas import tpu as pltpu
from jax.experimental.pallas import tpu_sc as plsc


def _build_gather(n_rows: int, d: int, n_table_rows: int):
    chunk = 16
    info = plsc.get_sparse_core_info()
    nc, ns = info.num_cores, info.num_subcores
    nw = nc * ns
    assert n_rows % nw == 0
    per_w = n_rows // nw
    assert per_w % chunk == 0
    n_chunks = per_w // chunk
    assert n_chunks % 2 == 0 and n_chunks >= 4

    mesh = plsc.VectorSubcoreMesh(core_axis_name="c", subcore_axis_name="s")

    @functools.partial(
        pl.kernel,
        mesh=mesh,
        out_type=jax.ShapeDtypeStruct((n_rows, d), jnp.float32),
        scratch_types=[
            pltpu.VMEM((per_w,), jnp.int32),
            pltpu.VMEM((chunk, d), jnp.float32),
            pltpu.VMEM((chunk, d), jnp.float32),
            pltpu.VMEM_SHARED((n_table_rows, d), jnp.float32),
            pltpu.SemaphoreType.DMA,
            pltpu.SemaphoreType.DMA,
            pltpu.SemaphoreType.DMA,
            pltpu.SemaphoreType.DMA,
        ],
    )
    def gather_kernel(idx_hbm, table_hbm, out_hbm, idx_v, rows_a, rows_b,
                      table_sp, fsem_a, fsem_b, osem_a, osem_b):
        wid = lax.axis_index("s") * nc + lax.axis_index("c")
        base = wid * per_w

        sid = lax.axis_index("s")
        rows_full = ((n_table_rows + ns - 1) // ns + 7) // 8 * 8
        rows_last = n_table_rows - (ns - 1) * rows_full
        assert 0 < rows_last <= rows_full

        @pl.when(sid < ns - 1)
        def _():
            pltpu.sync_copy(table_hbm.at[pl.ds(sid * rows_full, rows_full)],
                            table_sp.at[pl.ds(sid * rows_full, rows_full)])

        @pl.when(sid == ns - 1)
        def _():
            off = (ns - 1) * rows_full
            pltpu.sync_copy(table_hbm.at[pl.ds(off, rows_last)],
                            table_sp.at[pl.ds(off, rows_last)])

        pltpu.sync_copy(idx_hbm.at[pl.ds(base, per_w)], idx_v)
        plsc.subcore_barrier()

        bufs = ((rows_a, fsem_a, osem_a), (rows_b, fsem_b, osem_b))

        def fill(c, rows, fsem):
            vec = idx_v[pl.ds(c * chunk, chunk)]
            for l in range(chunk):
                pltpu.async_copy(table_sp.at[vec[l]], rows.at[l], fsem)
            pltpu.make_async_copy(table_hbm.at[pl.ds(0, chunk)], rows,
                                  fsem).wait()

        def out_slice(c):
            return out_hbm.at[pl.ds(base + c * chunk, chunk)]

        def start_out(c, rows, osem):
            pltpu.async_copy(rows, out_slice(c), osem)

        def wait_out(c, rows, osem):
            pltpu.make_async_copy(rows, out_slice(c), osem).wait()

        for b, (rows, fsem, osem) in enumerate(bufs):
            fill(b, rows, fsem)
            start_out(b, rows, osem)

        def body(p, carry):
            for b, (rows, fsem, osem) in enumerate(bufs):
                c = 2 * p + b
                wait_out(c - 2, rows, osem)
                fill(c, rows, fsem)
                start_out(c, rows, osem)
            return carry

        lax.fori_loop(1, n_chunks // 2, body, 0)

        for b, (rows, fsem, osem) in enumerate(bufs):
            c = n_chunks - 2 + b
            wait_out(c, rows, osem)

    return gather_kernel


def kernel(indices, bank_embedding_weight):
    b, s = indices.shape
    v, d = bank_embedding_weight.shape
    n = b * s
    flat = indices.reshape(n).astype(jnp.int32)
    out = _build_gather(n, d, n_table_rows=v)(flat, bank_embedding_weight)
    return out.reshape(b, s, d)
